# Initial kernel scaffold; baseline (speedup 1.0000x reference)
import jax
import jax.numpy as jnp
from jax.experimental import pallas as pl


def kernel(token_ids, table):
    raise NotImplementedError("write your pallas kernel here")


# speedup vs baseline: 9.1324x; 9.1324x over previous
import jax
import jax.numpy as jnp
from jax.experimental import pallas as pl
from jax.experimental.pallas import tpu as pltpu
from jax.experimental.pallas import tpu_sc as plsc

EMBED_DIM = 128
WINDOW = 256


def _gather_sc(table, flat_ids):
    num_indices = flat_ids.shape[0]
    ids2d = flat_ids.reshape(1, num_indices)
    mesh = plsc.VectorSubcoreMesh(core_axis_name="c", subcore_axis_name="s")

    @pl.kernel(
        out_type=jax.ShapeDtypeStruct((num_indices, EMBED_DIM), table.dtype),
        mesh=mesh,
    )
    def gather_kernel(table_hbm, ids_hbm, out_hbm):
        def body(ids_vmem, out_vmem):
            pltpu.sync_copy(table_hbm.at[ids_vmem.at[0]], out_vmem)

        pltpu.emit_pipeline(
            body,
            grid=(num_indices // WINDOW,),
            in_specs=[pl.BlockSpec((1, WINDOW), index_map=lambda i: (0, i))],
            out_specs=[pl.BlockSpec((WINDOW, EMBED_DIM), index_map=lambda i: (i, 0))],
            core_axis_name=("c", "s"),
            dimension_semantics=(pltpu.PARALLEL,),
        )(ids_hbm, out_hbm)

    return gather_kernel(table, ids2d)


def kernel(token_ids, table):
    batch, seq = token_ids.shape
    flat = token_ids.reshape(batch * seq).astype(jnp.int32)
    out = _gather_sc(table, flat)
    return out.reshape(batch, seq, EMBED_DIM)
